# baseline (speedup 1.0000x reference)
import functools
import math

import jax
import jax.numpy as jnp
from jax import lax
from jax.experimental import pallas as pl
from jax.experimental.pallas import tpu as pltpu
from jax.experimental.pallas import tpu_sc as plsc

D_MODEL = 64
SCALE_F = float(math.sqrt(D_MODEL))
CLEN = 128
LANES = 16

_info = plsc.get_sparse_core_info()
_NC = _info.num_cores
_NS = _info.num_subcores
_NW = _NC * _NS


@functools.lru_cache(maxsize=None)
def _make_sc_gather(n_chunks: int):
    rows_per_w = n_chunks * CLEN
    b_total = _NW * rows_per_w
    mesh = plsc.VectorSubcoreMesh(core_axis_name="c", subcore_axis_name="s")

    @functools.partial(
        pl.kernel,
        mesh=mesh,
        out_type=jax.ShapeDtypeStruct((b_total, D_MODEL), jnp.float32),
        scratch_types=[
            pltpu.VMEM((n_chunks, CLEN), jnp.int32),
            pltpu.VMEM((CLEN, D_MODEL), jnp.float32),
            pltpu.SemaphoreType.DMA,
        ],
        compiler_params=pltpu.CompilerParams(use_tc_tiling_on_sc=False),
    )
    def k(table_hbm, idx_hbm, out_hbm, idx_v, rows_v, sem):
        c = lax.axis_index("c")
        s = lax.axis_index("s")
        wid = s * _NC + c
        pltpu.sync_copy(idx_hbm.at[wid], idx_v)
        row0 = wid * rows_per_w

        def chunk_body(g, carry):
            pltpu.async_copy(table_hbm.at[idx_v.at[g]], rows_v, sem).wait()

            def scale_row(r, carry2):
                for kk in range(D_MODEL // LANES):
                    sl = pl.ds(kk * LANES, LANES)
                    rows_v[r, sl] = rows_v[r, sl] * SCALE_F
                return carry2

            lax.fori_loop(0, CLEN, scale_row, 0)
            pltpu.sync_copy(rows_v, out_hbm.at[pl.ds(row0 + g * CLEN, CLEN)])
            return carry

        lax.fori_loop(0, n_chunks, chunk_body, 0)

    return k


def kernel(x, table):
    b, t = x.shape
    n_total = b * t
    assert n_total % (_NW * CLEN) == 0
    n_chunks = n_total // (_NW * CLEN)
    idx = x.reshape(-1).astype(jnp.int32).reshape(_NW, n_chunks, CLEN)
    out = _make_sc_gather(n_chunks)(table, idx)
    return out.reshape(b, t, D_MODEL)


# speedup vs baseline: 1.2036x; 1.2036x over previous
import functools
import math

import jax
import jax.numpy as jnp
from jax import lax
from jax.experimental import pallas as pl
from jax.experimental.pallas import tpu as pltpu
from jax.experimental.pallas import tpu_sc as plsc

D_MODEL = 64
SCALE_F = float(math.sqrt(D_MODEL))
CLEN = 128
LANES = 16
NBUF = 4
RUNROLL = 8

_info = plsc.get_sparse_core_info()
_NC = _info.num_cores
_NS = _info.num_subcores
_NW = _NC * _NS


@functools.lru_cache(maxsize=None)
def _make_sc_gather(n_chunks: int):
    rows_per_w = n_chunks * CLEN
    b_total = _NW * rows_per_w
    mesh = plsc.VectorSubcoreMesh(core_axis_name="c", subcore_axis_name="s")

    @functools.partial(
        pl.kernel,
        mesh=mesh,
        out_type=jax.ShapeDtypeStruct((b_total, D_MODEL), jnp.float32),
        scratch_types=[
            pltpu.VMEM((n_chunks, CLEN), jnp.int32),
            pltpu.VMEM((NBUF, CLEN, D_MODEL), jnp.float32),
            pltpu.VMEM((NBUF, CLEN, D_MODEL), jnp.float32),
            pltpu.SemaphoreType.DMA((NBUF,)),
            pltpu.SemaphoreType.DMA((NBUF,)),
        ],
        compiler_params=pltpu.CompilerParams(use_tc_tiling_on_sc=False),
    )
    def k(table_hbm, idx_hbm, out_hbm, idx_v, gbuf, obuf, gsem, ssem):
        c = lax.axis_index("c")
        s = lax.axis_index("s")
        wid = s * _NC + c
        pltpu.sync_copy(idx_hbm.at[wid], idx_v)
        row0 = wid * rows_per_w

        for b in range(NBUF):
            pltpu.async_copy(table_hbm.at[idx_v.at[b]], gbuf.at[b], gsem.at[b])

        def outer(k0, carry):
            g0 = k0 * NBUF
            for b in range(NBUF):
                g = g0 + b
                pltpu.make_async_copy(
                    table_hbm.at[idx_v.at[g]], gbuf.at[b], gsem.at[b]
                ).wait()

                def scale_rows(r8, cc, b=b):
                    for dr in range(RUNROLL):
                        r = r8 * RUNROLL + dr
                        for kk in range(D_MODEL // LANES):
                            sl = pl.ds(kk * LANES, LANES)
                            obuf[b, r, sl] = gbuf[b, r, sl] * SCALE_F
                    return cc

                lax.fori_loop(0, CLEN // RUNROLL, scale_rows, 0)

                @pl.when(g0 > 0)
                def _wait_prev_store(b=b):
                    pltpu.make_async_copy(
                        obuf.at[b], out_hbm.at[pl.ds(row0, CLEN)], ssem.at[b]
                    ).wait()

                pltpu.make_async_copy(
                    obuf.at[b], out_hbm.at[pl.ds(row0 + g * CLEN, CLEN)],
                    ssem.at[b],
                ).start()

                @pl.when(g + NBUF < n_chunks)
                def _next_gather(g=g, b=b):
                    pltpu.async_copy(
                        table_hbm.at[idx_v.at[g + NBUF]], gbuf.at[b], gsem.at[b]
                    )
            return carry

        lax.fori_loop(0, n_chunks // NBUF, outer, 0)

        for b in range(NBUF):
            pltpu.make_async_copy(
                obuf.at[b], out_hbm.at[pl.ds(row0, CLEN)], ssem.at[b]
            ).wait()

    return k


def kernel(x, table):
    b, t = x.shape
    n_total = b * t
    assert n_total % (_NW * CLEN) == 0
    n_chunks = n_total // (_NW * CLEN)
    assert n_chunks % NBUF == 0 and CLEN % RUNROLL == 0
    idx = x.reshape(-1).astype(jnp.int32).reshape(_NW, n_chunks, CLEN)
    out = _make_sc_gather(n_chunks)(table, idx)
    return out.reshape(b, t, D_MODEL)
